# baseline (speedup 1.0000x reference)
import functools

import jax
import jax.numpy as jnp
from jax import lax
from jax.experimental import pallas as pl
from jax.experimental.pallas import tpu as pltpu
from jax.experimental.pallas import tpu_sc as plsc

N = 1_000_000
CR = 2000
NCHUNKS = N // CR
NW = 32
EPSILON = 1e-7

_mesh = plsc.VectorSubcoreMesh(core_axis_name="c", subcore_axis_name="s")


@functools.partial(
    pl.kernel,
    out_type=jax.ShapeDtypeStruct((NW, 48), jnp.float32),
    mesh=_mesh,
    compiler_params=pltpu.CompilerParams(
        needs_layout_passes=False, use_tc_tiling_on_sc=False),
    scratch_types=[
        pltpu.VMEM((CR, 4), jnp.float32),
        pltpu.VMEM((CR, 4), jnp.float32),
        pltpu.VMEM((CR,), jnp.int32),
        pltpu.VMEM((48,), jnp.float32),
    ],
)
def _loss_partials(t_hbm, r_hbm, lab_hbm, out_hbm, tv, rv, lv, accv):
    wid = lax.axis_index("s") * 2 + lax.axis_index("c")
    iota = lax.iota(jnp.int32, 16)
    row_rep = iota >> 2
    col = iota & 3
    zero = jnp.zeros((16,), jnp.float32)
    one = jnp.ones((16,), jnp.float32)

    nch = jnp.where(wid < (NCHUNKS % NW), NCHUNKS // NW + 1, NCHUNKS // NW)

    def chunk_body(k, carry):
        c = wid + k * NW
        pltpu.sync_copy(t_hbm.at[pl.ds(c * CR, CR), :], tv)
        pltpu.sync_copy(r_hbm.at[pl.ds(c * CR, CR), :], rv)
        pltpu.sync_copy(lab_hbm.at[pl.ds(c * CR, CR)], lv)

        def step(s, acc_a):
            ridx = s * 4 + row_rep
            t = plsc.load_gather(tv, [ridx, col])
            r = plsc.load_gather(rv, [ridx, col])
            labx = plsc.load_gather(lv, [ridx])
            x = t - r
            ax = jnp.abs(x)
            u = jnp.minimum(ax, 1.0)
            sl2 = u * (ax + ax - u)
            return acc_a + jnp.where(labx == 1, sl2, zero)

        acc_a = lax.fori_loop(0, CR // 4, step, carry[0])

        def lstep(q, vp):
            av, ap = vp
            lab16 = lv[pl.ds(q * 16, 16)]
            av = av + jnp.where(lab16 != -1, one, zero)
            ap = ap + jnp.where(lab16 == 1, one, zero)
            return av, ap

        acc_v, acc_p = lax.fori_loop(0, CR // 16, lstep, (carry[1], carry[2]))
        return acc_a, acc_v, acc_p

    acc_a, acc_v, acc_p = lax.fori_loop(0, nch, chunk_body, (zero, zero, zero))
    accv[pl.ds(0, 16)] = acc_a
    accv[pl.ds(16, 16)] = acc_v
    accv[pl.ds(32, 16)] = acc_p
    pltpu.sync_copy(accv, out_hbm.at[wid])


def kernel(rpn_bbox_targets, rpn_regression, rpn_labels):
    r2 = jnp.reshape(rpn_regression, (-1, 4))
    parts = _loss_partials(rpn_bbox_targets, r2, rpn_labels)
    a = 0.5 * jnp.sum(parts[:, 0:16])
    nvalid = jnp.sum(parts[:, 16:32])
    npos = jnp.sum(parts[:, 32:48])
    b = nvalid * EPSILON + npos
    return a / b


# speedup vs baseline: 1.2233x; 1.2233x over previous
import functools

import jax
import jax.numpy as jnp
from jax import lax
from jax.experimental import pallas as pl
from jax.experimental.pallas import tpu as pltpu
from jax.experimental.pallas import tpu_sc as plsc

N = 1_000_000
PR = N * 4 // 128
PRB = 128
MAINC = 244
TAILP = PR - MAINC * PRB
NW = 32
TAIL_W = 20
EPSILON = 1e-7

_mesh = plsc.VectorSubcoreMesh(core_axis_name="c", subcore_axis_name="s")


@functools.partial(
    pl.kernel,
    out_type=jax.ShapeDtypeStruct((NW, 128), jnp.float32),
    mesh=_mesh,
    scratch_types=[
        pltpu.VMEM((PRB, 128), jnp.float32),
        pltpu.VMEM((PRB, 128), jnp.float32),
        pltpu.VMEM((PRB * 32,), jnp.int32),
        pltpu.VMEM((TAILP, 128), jnp.float32),
        pltpu.VMEM((TAILP, 128), jnp.float32),
        pltpu.VMEM((TAILP * 32,), jnp.int32),
        pltpu.VMEM((128,), jnp.float32),
    ],
)
def _loss_partials(t_hbm, r_hbm, lab_hbm, tt_hbm, rt_hbm, out_hbm,
                   tv, rv, lv, tvt, rvt, lvt, accv):
    wid = lax.axis_index("s") * 2 + lax.axis_index("c")
    iota = lax.iota(jnp.int32, 16)
    row_rep = iota >> 2
    cidx = [jnp.asarray((j % 4) * 4, jnp.int32) + row_rep for j in range(8)]
    zero = jnp.zeros((16,), jnp.float32)
    one = jnp.ones((16,), jnp.float32)

    def make_prow(tref, rref, lref):
        def prow(p, acc):
            acc_a, acc_v, acc_p = acc
            lab_a = lref[pl.ds(p * 32, 16)]
            lab_b = lref[pl.ds(p * 32 + 16, 16)]
            wa = jnp.where(lab_a == 1, one, zero)
            wb = jnp.where(lab_b == 1, one, zero)
            acc_p = acc_p + wa + wb
            acc_v = (acc_v + jnp.where(lab_a != -1, one, zero)
                     + jnp.where(lab_b != -1, one, zero))
            for j in range(8):
                w16 = wa if j < 4 else wb
                wx = w16.at[cidx[j]].get(mode="promise_in_bounds")
                t = tref[p, pl.ds(j * 16, 16)]
                r = rref[p, pl.ds(j * 16, 16)]
                x = t - r
                ax = jnp.abs(x)
                u = jnp.minimum(ax, 1.0)
                s2 = u * (ax + ax - u)
                acc_a = acc_a + s2 * wx
            return acc_a, acc_v, acc_p
        return prow

    nch = jnp.where(wid < (MAINC % NW), MAINC // NW + 1, MAINC // NW)

    def chunk_body(k, carry):
        c = wid + k * NW
        pltpu.sync_copy(t_hbm.at[pl.ds(c * PRB, PRB), :], tv)
        pltpu.sync_copy(r_hbm.at[pl.ds(c * PRB, PRB), :], rv)
        pltpu.sync_copy(lab_hbm.at[pl.ds(c * PRB * 32, PRB * 32)], lv)
        return lax.fori_loop(0, PRB, make_prow(tv, rv, lv), carry)

    acc_a, acc_v, acc_p = lax.fori_loop(0, nch, chunk_body, (zero, zero, zero))
    accv[pl.ds(0, 16)] = acc_a
    accv[pl.ds(16, 16)] = acc_v
    accv[pl.ds(32, 16)] = acc_p

    @pl.when(wid == TAIL_W)
    def _tail():
        pltpu.sync_copy(tt_hbm, tvt)
        pltpu.sync_copy(rt_hbm, rvt)
        pltpu.sync_copy(lab_hbm.at[pl.ds(MAINC * PRB * 32, TAILP * 32)], lvt)
        cr = (accv[pl.ds(0, 16)], accv[pl.ds(16, 16)], accv[pl.ds(32, 16)])
        ta, tvv, tp = lax.fori_loop(0, TAILP, make_prow(tvt, rvt, lvt), cr)
        accv[pl.ds(0, 16)] = ta
        accv[pl.ds(16, 16)] = tvv
        accv[pl.ds(32, 16)] = tp
    accv[pl.ds(48, 16)] = zero
    accv[pl.ds(64, 16)] = zero
    accv[pl.ds(80, 16)] = zero
    accv[pl.ds(96, 16)] = zero
    accv[pl.ds(112, 16)] = zero
    pltpu.sync_copy(accv, out_hbm.at[wid])


def kernel(rpn_bbox_targets, rpn_regression, rpn_labels):
    t128 = jnp.reshape(rpn_bbox_targets, (PR, 128))
    r128 = jnp.reshape(rpn_regression, (PR, 128))
    t_tail = lax.slice(t128, (MAINC * PRB, 0), (PR, 128))
    r_tail = lax.slice(r128, (MAINC * PRB, 0), (PR, 128))
    parts = _loss_partials(t128, r128, rpn_labels, t_tail, r_tail)
    a = 0.5 * jnp.sum(parts[:, 0:16])
    nvalid = jnp.sum(parts[:, 16:32])
    npos = jnp.sum(parts[:, 32:48])
    b = nvalid * EPSILON + npos
    return a / b


# speedup vs baseline: 15.4659x; 12.6430x over previous
import functools

import jax
import jax.numpy as jnp
from jax import lax
from jax.experimental import pallas as pl
from jax.experimental.pallas import tpu as pltpu
from jax.experimental.pallas import tpu_sc as plsc

N = 1_000_000
NB = N // 128
NMAIN = NB * 128
CB = 36
NCHUNKS = NB // CB
NW = 32
TAIL_W = 25
EPSILON = 1e-7

_mesh = plsc.VectorSubcoreMesh(core_axis_name="c", subcore_axis_name="s")


@functools.partial(
    pl.kernel,
    out_type=jax.ShapeDtypeStruct((NW, 128), jnp.float32),
    mesh=_mesh,
    compiler_params=pltpu.CompilerParams(use_tc_tiling_on_sc=False),
    scratch_types=[
        pltpu.VMEM((CB * 4, 128), jnp.float32),
        pltpu.VMEM((CB * 4, 128), jnp.float32),
        pltpu.VMEM((CB * 128,), jnp.int32),
        pltpu.VMEM((4, 64), jnp.float32),
        pltpu.VMEM((4, 64), jnp.float32),
        pltpu.VMEM((64,), jnp.int32),
        pltpu.VMEM((128,), jnp.float32),
    ],
)
def _loss_partials(t_hbm, r_hbm, lab_hbm, tt_hbm, rt_hbm, out_hbm,
                   tv, rv, lv, tvt, rvt, lvt, accv):
    wid = lax.axis_index("s") * 2 + lax.axis_index("c")
    zero = jnp.zeros((16,), jnp.float32)
    one = jnp.ones((16,), jnp.float32)

    def smooth2(t, r):
        x = t - r
        ax = jnp.abs(x)
        u = jnp.minimum(ax, 1.0)
        return u * (ax + ax - u)

    nch = jnp.where(wid < (NCHUNKS % NW), NCHUNKS // NW + 1, NCHUNKS // NW)

    def chunk_body(k, carry):
        c = wid + k * NW
        pltpu.sync_copy(t_hbm.at[pl.ds(c * CB * 4, CB * 4), :], tv)
        pltpu.sync_copy(r_hbm.at[pl.ds(c * CB * 4, CB * 4), :], rv)
        pltpu.sync_copy(lab_hbm.at[pl.ds(c * CB * 128, CB * 128)], lv)

        def block_body(b, acc):
            acc_a, acc_v, acc_p = acc
            for kk in range(8):
                lab16 = lv[pl.ds(b * 128 + kk * 16, 16)]
                w = jnp.where(lab16 == 1, one, zero)
                acc_p = acc_p + w
                acc_v = acc_v + jnp.where(lab16 != -1, one, zero)
                for j in range(4):
                    t = tv[b * 4 + j, pl.ds(kk * 16, 16)]
                    r = rv[b * 4 + j, pl.ds(kk * 16, 16)]
                    acc_a = acc_a + smooth2(t, r) * w
            return acc_a, acc_v, acc_p

        return lax.fori_loop(0, CB, block_body, carry)

    acc_a, acc_v, acc_p = lax.fori_loop(0, nch, chunk_body, (zero, zero, zero))
    accv[pl.ds(0, 16)] = acc_a
    accv[pl.ds(16, 16)] = acc_v
    accv[pl.ds(32, 16)] = acc_p
    accv[pl.ds(48, 16)] = zero
    accv[pl.ds(64, 16)] = zero
    accv[pl.ds(80, 16)] = zero
    accv[pl.ds(96, 16)] = zero
    accv[pl.ds(112, 16)] = zero

    @pl.when(wid == TAIL_W)
    def _tail():
        pltpu.sync_copy(tt_hbm, tvt)
        pltpu.sync_copy(rt_hbm, rvt)
        pltpu.sync_copy(lab_hbm.at[pl.ds(NMAIN, 64)], lvt)
        ta = accv[pl.ds(0, 16)]
        tp = accv[pl.ds(32, 16)]
        tvv = accv[pl.ds(16, 16)]
        for kk in range(4):
            lab16 = lvt[pl.ds(kk * 16, 16)]
            w = jnp.where(lab16 == 1, one, zero)
            tp = tp + w
            tvv = tvv + jnp.where(lab16 != -1, one, zero)
            for j in range(4):
                t = tvt[j, pl.ds(kk * 16, 16)]
                r = rvt[j, pl.ds(kk * 16, 16)]
                ta = ta + smooth2(t, r) * w
        accv[pl.ds(0, 16)] = ta
        accv[pl.ds(16, 16)] = tvv
        accv[pl.ds(32, 16)] = tp

    pltpu.sync_copy(accv, out_hbm.at[wid])


def _as_blocked(x):
    m = lax.slice(x, (0, 0), (NMAIN, 4))
    return jnp.reshape(jnp.transpose(jnp.reshape(m, (NB, 128, 4)), (0, 2, 1)),
                       (NB * 4, 128))


def kernel(rpn_bbox_targets, rpn_regression, rpn_labels):
    tb = _as_blocked(rpn_bbox_targets)
    rb = _as_blocked(rpn_regression)
    t_tail = jnp.transpose(lax.slice(rpn_bbox_targets, (NMAIN, 0), (N, 4)))
    r_tail = jnp.transpose(lax.slice(rpn_regression, (NMAIN, 0), (N, 4)))
    parts = _loss_partials(tb, rb, rpn_labels, t_tail, r_tail)
    a = 0.5 * jnp.sum(parts[:, 0:16])
    nvalid = jnp.sum(parts[:, 16:32])
    npos = jnp.sum(parts[:, 32:48])
    b = nvalid * EPSILON + npos
    return a / b


# speedup vs baseline: 21.8899x; 1.4154x over previous
import functools

import jax
import jax.numpy as jnp
from jax import lax
from jax.experimental import pallas as pl
from jax.experimental.pallas import tpu as pltpu
from jax.experimental.pallas import tpu_sc as plsc

N = 1_000_000
CA = 4000
NCHUNKS = N // CA
NW = 32
EPSILON = 1e-7

_mesh = plsc.VectorSubcoreMesh(core_axis_name="c", subcore_axis_name="s")


@functools.partial(
    pl.kernel,
    out_type=jax.ShapeDtypeStruct((NW, 128), jnp.float32),
    mesh=_mesh,
    compiler_params=pltpu.CompilerParams(use_tc_tiling_on_sc=False),
    scratch_types=[
        pltpu.VMEM((4, CA), jnp.float32),
        pltpu.VMEM((4, CA), jnp.float32),
        pltpu.VMEM((CA,), jnp.int32),
        pltpu.VMEM((128,), jnp.float32),
    ],
)
def _loss_partials(t_hbm, r_hbm, lab_hbm, out_hbm, tv, rv, lv, accv):
    wid = lax.axis_index("s") * 2 + lax.axis_index("c")
    zero = jnp.zeros((16,), jnp.float32)
    one = jnp.ones((16,), jnp.float32)

    nch = jnp.where(wid < (NCHUNKS % NW), NCHUNKS // NW + 1, NCHUNKS // NW)

    def chunk_body(k, carry):
        c = wid + k * NW
        pltpu.sync_copy(t_hbm.at[:, pl.ds(c * CA, CA)], tv)
        pltpu.sync_copy(r_hbm.at[:, pl.ds(c * CA, CA)], rv)
        pltpu.sync_copy(lab_hbm.at[pl.ds(c * CA, CA)], lv)

        def group_body(g, acc):
            acc_a, acc_v, acc_p = acc
            lab16 = lv[pl.ds(g * 16, 16)]
            w = jnp.where(lab16 == 1, one, zero)
            acc_p = acc_p + w
            acc_v = acc_v + jnp.where(lab16 != -1, one, zero)
            for j in range(4):
                t = tv[j, pl.ds(g * 16, 16)]
                r = rv[j, pl.ds(g * 16, 16)]
                x = t - r
                ax = jnp.abs(x)
                u = jnp.minimum(ax, 1.0)
                acc_a = acc_a + (u * (ax + ax - u)) * w
            return acc_a, acc_v, acc_p

        return lax.fori_loop(0, CA // 16, group_body, carry)

    acc_a, acc_v, acc_p = lax.fori_loop(0, nch, chunk_body, (zero, zero, zero))
    accv[pl.ds(0, 16)] = acc_a
    accv[pl.ds(16, 16)] = acc_v
    accv[pl.ds(32, 16)] = acc_p
    accv[pl.ds(48, 16)] = zero
    accv[pl.ds(64, 16)] = zero
    accv[pl.ds(80, 16)] = zero
    accv[pl.ds(96, 16)] = zero
    accv[pl.ds(112, 16)] = zero
    pltpu.sync_copy(accv, out_hbm.at[wid])


def kernel(rpn_bbox_targets, rpn_regression, rpn_labels):
    tp = jnp.transpose(rpn_bbox_targets)
    rp = jnp.transpose(rpn_regression)
    parts = _loss_partials(tp, rp, rpn_labels)
    a = 0.5 * jnp.sum(parts[:, 0:16])
    nvalid = jnp.sum(parts[:, 16:32])
    npos = jnp.sum(parts[:, 32:48])
    b = nvalid * EPSILON + npos
    return a / b


# speedup vs baseline: 26.0925x; 1.1920x over previous
import functools

import jax
import jax.numpy as jnp
from jax import lax
from jax.experimental import pallas as pl
from jax.experimental.pallas import tpu as pltpu
from jax.experimental.pallas import tpu_sc as plsc

N = 1_000_000
CA = 4000
NCHUNKS = N // CA
NW = 32
EPSILON = 1e-7

_mesh = plsc.VectorSubcoreMesh(core_axis_name="c", subcore_axis_name="s")


@functools.partial(
    pl.kernel,
    out_type=jax.ShapeDtypeStruct((NW, 128), jnp.float32),
    mesh=_mesh,
    compiler_params=pltpu.CompilerParams(use_tc_tiling_on_sc=False),
    scratch_types=[
        pltpu.VMEM((2, 4, CA), jnp.float32),
        pltpu.VMEM((2, 4, CA), jnp.float32),
        pltpu.VMEM((2, CA), jnp.int32),
        pltpu.VMEM((128,), jnp.float32),
        pltpu.SemaphoreType.DMA((2,)),
        pltpu.SemaphoreType.DMA((2,)),
        pltpu.SemaphoreType.DMA((2,)),
    ],
)
def _loss_partials(t_hbm, r_hbm, lab_hbm, out_hbm,
                   tv, rv, lv, accv, tsem, rsem, lsem):
    wid = lax.axis_index("s") * 2 + lax.axis_index("c")
    zero = jnp.zeros((16,), jnp.float32)
    one = jnp.ones((16,), jnp.float32)

    nch = jnp.where(wid < (NCHUNKS % NW), NCHUNKS // NW + 1, NCHUNKS // NW)

    def copies(k):
        c = wid + k * NW
        buf = lax.rem(k, 2)
        return (
            pltpu.make_async_copy(t_hbm.at[:, pl.ds(c * CA, CA)],
                                  tv.at[buf], tsem.at[buf]),
            pltpu.make_async_copy(r_hbm.at[:, pl.ds(c * CA, CA)],
                                  rv.at[buf], rsem.at[buf]),
            pltpu.make_async_copy(lab_hbm.at[pl.ds(c * CA, CA)],
                                  lv.at[buf], lsem.at[buf]),
        )

    def start(k):
        for cp in copies(k):
            cp.start()

    start(0)

    def chunk_body(k, carry):
        @pl.when(k + 1 < nch)
        def _prefetch():
            start(k + 1)

        for cp in copies(k):
            cp.wait()
        buf = lax.rem(k, 2)

        def group_body(g2, acc):
            acc_a, acc_p, acc_s = acc
            for half in range(2):
                g = g2 * 2 + half
                lab16 = lv[buf, pl.ds(g * 16, 16)]
                w = jnp.where(lab16 == 1, one, zero)
                acc_p = acc_p + w
                acc_s = acc_s + lab16.astype(jnp.float32)
                for j in range(4):
                    t = tv[buf, j, pl.ds(g * 16, 16)]
                    r = rv[buf, j, pl.ds(g * 16, 16)]
                    x = t - r
                    ax = jnp.abs(x)
                    u = jnp.minimum(ax, 1.0)
                    acc_a = acc_a + (u * (ax + ax - u)) * w
            return acc_a, acc_p, acc_s

        return lax.fori_loop(0, CA // 32, group_body, carry)

    acc_a, acc_p, acc_s = lax.fori_loop(0, nch, chunk_body, (zero, zero, zero))
    accv[pl.ds(0, 16)] = acc_a
    accv[pl.ds(16, 16)] = acc_p
    accv[pl.ds(32, 16)] = acc_s
    accv[pl.ds(48, 16)] = zero
    accv[pl.ds(64, 16)] = zero
    accv[pl.ds(80, 16)] = zero
    accv[pl.ds(96, 16)] = zero
    accv[pl.ds(112, 16)] = zero
    pltpu.sync_copy(accv, out_hbm.at[wid])


def kernel(rpn_bbox_targets, rpn_regression, rpn_labels):
    tp = jnp.transpose(rpn_bbox_targets)
    rp = jnp.transpose(rpn_regression)
    parts = _loss_partials(tp, rp, rpn_labels)
    a = 0.5 * jnp.sum(parts[:, 0:16])
    npos = jnp.sum(parts[:, 16:32])
    lsum = jnp.sum(parts[:, 32:48])
    nvalid = jnp.float32(N) - npos + lsum
    b = nvalid * EPSILON + npos
    return a / b


# speedup vs baseline: 28.1211x; 1.0777x over previous
import functools

import jax
import jax.numpy as jnp
from jax import lax
from jax.experimental import pallas as pl
from jax.experimental.pallas import tpu as pltpu
from jax.experimental.pallas import tpu_sc as plsc

N = 1_000_000
CA = 4000
H1 = 496_000
H2 = N - H1
NW = 32
EPSILON = 1e-7

_mesh = plsc.VectorSubcoreMesh(core_axis_name="c", subcore_axis_name="s")


def _make_half(nanch, lab_off):
    nchunks = nanch // CA

    @functools.partial(
        pl.kernel,
        out_type=jax.ShapeDtypeStruct((NW, 128), jnp.float32),
        mesh=_mesh,
        compiler_params=pltpu.CompilerParams(use_tc_tiling_on_sc=False),
        scratch_types=[
            pltpu.VMEM((2, 4, CA), jnp.float32),
            pltpu.VMEM((2, 4, CA), jnp.float32),
            pltpu.VMEM((2, CA), jnp.int32),
            pltpu.VMEM((128,), jnp.float32),
            pltpu.SemaphoreType.DMA((2,)),
            pltpu.SemaphoreType.DMA((2,)),
            pltpu.SemaphoreType.DMA((2,)),
        ],
    )
    def _half(t_hbm, r_hbm, lab_hbm, out_hbm,
              tv, rv, lv, accv, tsem, rsem, lsem):
        wid = lax.axis_index("s") * 2 + lax.axis_index("c")
        zero = jnp.zeros((16,), jnp.float32)
        one = jnp.ones((16,), jnp.float32)

        nch = jnp.where(wid < (nchunks % NW), nchunks // NW + 1,
                        nchunks // NW)

        def copies(k):
            c = wid + k * NW
            buf = lax.rem(k, 2)
            return (
                pltpu.make_async_copy(t_hbm.at[:, pl.ds(c * CA, CA)],
                                      tv.at[buf], tsem.at[buf]),
                pltpu.make_async_copy(r_hbm.at[:, pl.ds(c * CA, CA)],
                                      rv.at[buf], rsem.at[buf]),
                pltpu.make_async_copy(lab_hbm.at[pl.ds(lab_off + c * CA, CA)],
                                      lv.at[buf], lsem.at[buf]),
            )

        def start(k):
            for cp in copies(k):
                cp.start()

        start(0)

        def chunk_body(k, carry):
            @pl.when(k + 1 < nch)
            def _prefetch():
                start(k + 1)

            for cp in copies(k):
                cp.wait()
            buf = lax.rem(k, 2)

            def group_body(g2, acc):
                acc_a, acc_p, acc_s = acc
                for half in range(2):
                    g = g2 * 2 + half
                    lab16 = lv[buf, pl.ds(g * 16, 16)]
                    w = jnp.where(lab16 == 1, one, zero)
                    acc_p = acc_p + w
                    acc_s = acc_s + lab16.astype(jnp.float32)
                    for j in range(4):
                        t = tv[buf, j, pl.ds(g * 16, 16)]
                        r = rv[buf, j, pl.ds(g * 16, 16)]
                        x = t - r
                        ax = jnp.abs(x)
                        u = jnp.minimum(ax, 1.0)
                        acc_a = acc_a + (u * (ax + ax - u)) * w
                return acc_a, acc_p, acc_s

            return lax.fori_loop(0, CA // 32, group_body, carry)

        acc_a, acc_p, acc_s = lax.fori_loop(0, nch, chunk_body,
                                            (zero, zero, zero))
        accv[pl.ds(0, 16)] = acc_a
        accv[pl.ds(16, 16)] = acc_p
        accv[pl.ds(32, 16)] = acc_s
        accv[pl.ds(48, 16)] = zero
        accv[pl.ds(64, 16)] = zero
        accv[pl.ds(80, 16)] = zero
        accv[pl.ds(96, 16)] = zero
        accv[pl.ds(112, 16)] = zero
        pltpu.sync_copy(accv, out_hbm.at[wid])

    return _half


_half_a = _make_half(H1, 0)
_half_b = _make_half(H2, H1)


def kernel(rpn_bbox_targets, rpn_regression, rpn_labels):
    ta = jnp.transpose(lax.slice(rpn_bbox_targets, (0, 0), (H1, 4)))
    ra = jnp.transpose(lax.slice(rpn_regression, (0, 0), (H1, 4)))
    tb = jnp.transpose(lax.slice(rpn_bbox_targets, (H1, 0), (N, 4)))
    rb = jnp.transpose(lax.slice(rpn_regression, (H1, 0), (N, 4)))
    parts = (_half_a(ta, ra, rpn_labels) + _half_b(tb, rb, rpn_labels))
    a = 0.5 * jnp.sum(parts[:, 0:16])
    npos = jnp.sum(parts[:, 16:32])
    lsum = jnp.sum(parts[:, 32:48])
    nvalid = jnp.float32(N) - npos + lsum
    b = nvalid * EPSILON + npos
    return a / b


# speedup vs baseline: 30.8702x; 1.0978x over previous
import functools

import jax
import jax.numpy as jnp
from jax import lax
from jax.experimental import pallas as pl
from jax.experimental.pallas import tpu as pltpu
from jax.experimental.pallas import tpu_sc as plsc

N = 1_000_000
CA = 4000
_BOUNDS = (0, 248_000, 496_000, 744_000, N)
NW = 32
EPSILON = 1e-7

_mesh = plsc.VectorSubcoreMesh(core_axis_name="c", subcore_axis_name="s")


def _make_part(nanch, lab_off):
    nchunks = nanch // CA

    @functools.partial(
        pl.kernel,
        out_type=jax.ShapeDtypeStruct((NW, 128), jnp.float32),
        mesh=_mesh,
        compiler_params=pltpu.CompilerParams(use_tc_tiling_on_sc=False),
        scratch_types=[
            pltpu.VMEM((2, 4, CA), jnp.float32),
            pltpu.VMEM((2, 4, CA), jnp.float32),
            pltpu.VMEM((2, CA), jnp.int32),
            pltpu.VMEM((128,), jnp.float32),
            pltpu.SemaphoreType.DMA((2,)),
            pltpu.SemaphoreType.DMA((2,)),
            pltpu.SemaphoreType.DMA((2,)),
        ],
    )
    def _part(t_hbm, r_hbm, lab_hbm, out_hbm,
              tv, rv, lv, accv, tsem, rsem, lsem):
        wid = lax.axis_index("s") * 2 + lax.axis_index("c")
        zero = jnp.zeros((16,), jnp.float32)
        one = jnp.ones((16,), jnp.float32)

        nch = jnp.where(wid < (nchunks % NW), nchunks // NW + 1,
                        nchunks // NW)

        def copies(k):
            c = wid + k * NW
            buf = lax.rem(k, 2)
            return (
                pltpu.make_async_copy(t_hbm.at[:, pl.ds(c * CA, CA)],
                                      tv.at[buf], tsem.at[buf]),
                pltpu.make_async_copy(r_hbm.at[:, pl.ds(c * CA, CA)],
                                      rv.at[buf], rsem.at[buf]),
                pltpu.make_async_copy(lab_hbm.at[pl.ds(lab_off + c * CA, CA)],
                                      lv.at[buf], lsem.at[buf]),
            )

        def start(k):
            for cp in copies(k):
                cp.start()

        start(0)

        def chunk_body(k, carry):
            @pl.when(k + 1 < nch)
            def _prefetch():
                start(k + 1)

            for cp in copies(k):
                cp.wait()
            buf = lax.rem(k, 2)

            def group_body(g2, acc):
                acc_a, acc_p, acc_s = acc
                for half in range(2):
                    g = g2 * 2 + half
                    lab16 = lv[buf, pl.ds(g * 16, 16)]
                    w = jnp.where(lab16 == 1, one, zero)
                    acc_p = acc_p + w
                    acc_s = acc_s + lab16.astype(jnp.float32)
                    for j in range(4):
                        t = tv[buf, j, pl.ds(g * 16, 16)]
                        r = rv[buf, j, pl.ds(g * 16, 16)]
                        x = t - r
                        ax = jnp.abs(x)
                        u = jnp.minimum(ax, 1.0)
                        acc_a = acc_a + (u * (ax + ax - u)) * w
                return acc_a, acc_p, acc_s

            return lax.fori_loop(0, CA // 32, group_body, carry)

        acc_a, acc_p, acc_s = lax.fori_loop(0, nch, chunk_body,
                                            (zero, zero, zero))
        accv[pl.ds(0, 16)] = acc_a
        accv[pl.ds(16, 16)] = acc_p
        accv[pl.ds(32, 16)] = acc_s
        accv[pl.ds(48, 16)] = zero
        accv[pl.ds(64, 16)] = zero
        accv[pl.ds(80, 16)] = zero
        accv[pl.ds(96, 16)] = zero
        accv[pl.ds(112, 16)] = zero
        pltpu.sync_copy(accv, out_hbm.at[wid])

    return _part


_parts = [_make_part(_BOUNDS[i + 1] - _BOUNDS[i], _BOUNDS[i])
          for i in range(4)]


def kernel(rpn_bbox_targets, rpn_regression, rpn_labels):
    parts = None
    for i in range(4):
        lo, hi = _BOUNDS[i], _BOUNDS[i + 1]
        tq = jnp.transpose(lax.slice(rpn_bbox_targets, (lo, 0), (hi, 4)))
        rq = jnp.transpose(lax.slice(rpn_regression, (lo, 0), (hi, 4)))
        p = _parts[i](tq, rq, rpn_labels)
        parts = p if parts is None else parts + p
    a = 0.5 * jnp.sum(parts[:, 0:16])
    npos = jnp.sum(parts[:, 16:32])
    lsum = jnp.sum(parts[:, 32:48])
    nvalid = jnp.float32(N) - npos + lsum
    b = nvalid * EPSILON + npos
    return a / b
